# baseline (speedup 1.0000x reference)
import functools

import jax
import jax.numpy as jnp
from jax import lax
from jax.experimental import pallas as pl
from jax.experimental.pallas import tpu as pltpu
from jax.experimental.pallas import tpu_sc as plsc

N = 10000
E = 320000
D = 128

NC = 2
NS = 16
NP = 10240
HP = NP // NC
RPT = HP // NS
EPT = E // NS
C = 80
NCHUNK = EPT // C
ZR = 64
DR = HP // D

_GDN = lax.GatherDimensionNumbers(
    offset_dims=(), collapsed_slice_dims=(0,), start_index_map=(0,))


def _lane_bcast(v, j):
    idx = jnp.full((16, 1), j, dtype=jnp.int32)
    return lax.gather(v, idx, _GDN, (1,),
                      mode=lax.GatherScatterMode.PROMISE_IN_BOUNDS)


_lane_bcast_i32 = _lane_bcast


def _make_sc_aggregate(with_deg: bool):
    Cc = 80
    NCH = EPT // Cc
    NB = 3
    NMAIN = (NCH // NB) * NB

    mesh = plsc.VectorSubcoreMesh(core_axis_name="c", subcore_axis_name="s")
    out_type = jax.ShapeDtypeStruct((NP, D), jnp.float32)
    if with_deg:
        out_type = (out_type, jax.ShapeDtypeStruct((NC * DR, D), jnp.float32))
    scratch = [
        pltpu.VMEM((EPT,), jnp.int32),
        pltpu.VMEM((EPT,), jnp.float32),
        pltpu.VMEM((ZR, D), jnp.float32),
        pltpu.VMEM_SHARED((HP + 8, D), jnp.float32),
    ]
    for _ in range(NB):
        scratch += [
            pltpu.VMEM((Cc,), jnp.int32),
            pltpu.VMEM((Cc,), jnp.int32),
            pltpu.VMEM((Cc, D), jnp.float32),
            pltpu.SemaphoreType.DMA,
            pltpu.SemaphoreType.DMA,
            pltpu.SemaphoreType.DMA,
        ]
    if with_deg:
        scratch += [pltpu.VMEM_SHARED((DR + 8, D), jnp.float32)]
        for _ in range(NB):
            scratch += [
                pltpu.VMEM((Cc,), jnp.int32),
                pltpu.VMEM((Cc, D), jnp.float32),
            ]

    @functools.partial(pl.kernel, out_type=out_type, mesh=mesh,
                       scratch_types=scratch)
    def agg(src_hbm, dst_hbm, w_hbm, x_hbm, *rest):
        if with_deg:
            out_hbm, deg_hbm = rest[0], rest[1]
            rest = rest[2:]
        else:
            out_hbm = rest[0]
            rest = rest[1:]
        dbuf, wbig, zb = rest[0], rest[1], rest[2]
        acc = rest[3]
        slots = []
        k = 4
        for _ in range(NB):
            slots.append(rest[k:k + 6])
            k += 6
        if with_deg:
            dacc = rest[k]
            k += 1
            dslots = []
            for _ in range(NB):
                dslots.append(rest[k:k + 2])
                k += 2

        c = lax.axis_index("c")
        s = lax.axis_index("s")
        lo = c * HP
        ebase = s * EPT

        pltpu.sync_copy(dst_hbm.at[pl.ds(ebase, EPT)], dbuf)
        pltpu.sync_copy(w_hbm.at[pl.ds(ebase, EPT)], wbig)

        def fill(i, _):
            for q in range(D // 16):
                zb[i, pl.ds(q * 16, 16)] = jnp.zeros((16,), jnp.float32)
            return 0
        lax.fori_loop(0, ZR, fill, 0)

        base = pl.multiple_of(s * RPT, ZR)
        for k2 in range(RPT // ZR):
            r0 = pl.multiple_of(base + k2 * ZR, ZR)
            pltpu.sync_copy(zb, acc.at[pl.ds(r0, ZR)])
        if with_deg:
            @pl.when(s == 0)
            def _():
                pltpu.sync_copy(zb.at[pl.ds(0, DR + 8)], dacc)
        plsc.subcore_barrier()

        def eoff(t):
            return pl.multiple_of(ebase + t * Cc, 8)

        def issue_sidx(t, sl):
            pltpu.async_copy(src_hbm.at[pl.ds(eoff(t), Cc)], sl[0], sl[3])

        def wait_sidx(sl):
            pltpu.make_async_copy(src_hbm.at[pl.ds(ebase, Cc)], sl[0],
                                  sl[3]).wait()

        def issue_gather(sl):
            pltpu.async_copy(x_hbm.at[sl[0]], sl[2], sl[4])

        def wait_gather(sl):
            pltpu.make_async_copy(x_hbm.at[sl[0]], sl[2], sl[4]).wait()

        def issue_scatter(p, sl):
            pltpu.async_copy(sl[2], acc.at[sl[1]], sl[5], add=True)
            if with_deg:
                pltpu.async_copy(dslots[p][1], dacc.at[dslots[p][0]], sl[5],
                                 add=True)

        def wait_scatter(p, sl):
            pltpu.make_async_copy(sl[2], acc.at[sl[1]], sl[5]).wait()
            if with_deg:
                pltpu.make_async_copy(dslots[p][1], dacc.at[dslots[p][0]],
                                      sl[5]).wait()

        def scale(t, p, sl):
            toff = pl.multiple_of(t * Cc, 8)
            didx, rows = sl[1], sl[2]
            for g in range(Cc // 16):
                sl16 = pl.ds(g * 16, 16)
                goff = pl.ds(toff + g * 16, 16)
                rel = dbuf[goff] - lo
                inb = (rel >= 0) & (rel < HP)
                didx[sl16] = jnp.where(inb, rel, HP)
                wv = wbig[goff]
                if with_deg:
                    r2 = jnp.where(inb, rel, 0)
                    dslots[p][0][sl16] = jnp.where(inb, r2 >> 7, DR)
                    dlane = r2 & (D - 1)
                for j in range(16):
                    e = g * 16 + j
                    wb = _lane_bcast(wv, j)
                    for q in range(D // 16):
                        sq = pl.ds(q * 16, 16)
                        rows[e, sq] = rows[e, sq] * wb
                    if with_deg:
                        dl = _lane_bcast_i32(dlane, j)
                        for q in range(D // 16):
                            io = lax.iota(jnp.int32, 16) + (q * 16)
                            dslots[p][1][e, pl.ds(q * 16, 16)] = jnp.where(
                                io == dl, 1.0, 0.0)

        for tt in range(NB):
            issue_sidx(tt, slots[tt])
        wait_sidx(slots[0])
        issue_gather(slots[0])

        def body(i, _):
            for p in range(NB):
                t = NB * i + p
                pn = (p + 1) % NB
                if p >= 2:
                    wait_scatter(pn, slots[pn])
                else:
                    @pl.when(i > 0)
                    def _():
                        wait_scatter(pn, slots[pn])
                wait_sidx(slots[pn])
                issue_gather(slots[pn])
                wait_gather(slots[p])

                @pl.when(t + NB < NCH)
                def _():
                    issue_sidx(t + NB, slots[p])
                scale(t, p, slots[p])
                issue_scatter(p, slots[p])
            return 0
        lax.fori_loop(0, NMAIN // NB, body, 0)

        for t in range(NMAIN, NCH):
            p = t % NB
            pn = (p + 1) % NB
            wait_scatter(pn, slots[pn])
            if t + 1 < NCH:
                wait_sidx(slots[pn])
                issue_gather(slots[pn])
            wait_gather(slots[p])
            scale(t, p, slots[p])
            issue_scatter(p, slots[p])

        for t in range(NCH - 2, NCH):
            wait_scatter(t % NB, slots[t % NB])
        plsc.subcore_barrier()

        for k3 in range(RPT // ZR):
            r0 = pl.multiple_of(base + k3 * ZR, ZR)
            go = lo + r0
            pltpu.sync_copy(acc.at[pl.ds(r0, ZR)], out_hbm.at[pl.ds(go, ZR)])
        if with_deg:
            @pl.when(s == 0)
            def _():
                pltpu.sync_copy(dacc.at[pl.ds(0, DR)],
                                deg_hbm.at[pl.ds(c * DR, DR)])

    return agg


_sc_aggregate_deg = _make_sc_aggregate(True)
_sc_aggregate = _make_sc_aggregate(False)

EB = 3200


def _tc_deg_body(dst_ref, o_ref):
    i = pl.program_id(0)

    @pl.when(i == 0)
    def _():
        o_ref[...] = jnp.zeros_like(o_ref)

    d = dst_ref[0, 0, :]
    dl = (d & (D - 1))[None, :]
    dr = (d >> 7)[None, :]
    iol = lax.broadcasted_iota(jnp.int32, (D, EB), 0)
    ior = lax.broadcasted_iota(jnp.int32, (NC * DR, EB), 0)
    lhs = jnp.maximum(1 - jnp.abs(ior - dr), 0).astype(jnp.bfloat16)
    rhs = jnp.maximum(1 - jnp.abs(iol - dl), 0).astype(jnp.bfloat16)
    o_ref[...] += lax.dot_general(
        lhs, rhs, (((1,), (1,)), ((), ())),
        preferred_element_type=jnp.float32)


def _tc_deg(dst):
    dst3 = dst.reshape(E // EB, 1, EB)
    return pl.pallas_call(
        _tc_deg_body,
        grid=(E // EB,),
        in_specs=[pl.BlockSpec((1, 1, EB), lambda i: (i, 0, 0))],
        out_specs=pl.BlockSpec((NC * DR, D), lambda i: (0, 0)),
        out_shape=jax.ShapeDtypeStruct((NC * DR, D), jnp.float32),
    )(dst3)


R = 1000


def _tc_dense_body(act, x_ref, agg_ref, deg_ref, ws_ref, wn_ref, b_ref, o_ref):
    h_n = agg_ref[...] / jnp.maximum(deg_ref[...], 1.0)
    y = (jnp.dot(x_ref[...], ws_ref[...], preferred_element_type=jnp.float32,
                 precision=lax.Precision.HIGHEST)
         + jnp.dot(h_n, wn_ref[...], preferred_element_type=jnp.float32,
                   precision=lax.Precision.HIGHEST)
         + b_ref[...])
    if act:
        y = jnp.maximum(y, 0.0)
        n2 = jnp.sum(y * y, axis=1, keepdims=True)
        y = y * lax.rsqrt(jnp.maximum(n2, 1e-24))
    o_ref[...] = y


def _tc_dense(x, agg, deg, wself_t, wneigh_t, b, act):
    dout = wself_t.shape[1]
    return pl.pallas_call(
        functools.partial(_tc_dense_body, act),
        grid=(N // R,),
        in_specs=[
            pl.BlockSpec((R, D), lambda i: (i, 0)),
            pl.BlockSpec((R, D), lambda i: (i, 0)),
            pl.BlockSpec((R, 1), lambda i: (i, 0)),
            pl.BlockSpec((D, dout), lambda i: (0, 0)),
            pl.BlockSpec((D, dout), lambda i: (0, 0)),
            pl.BlockSpec((1, dout), lambda i: (0, 0)),
        ],
        out_specs=pl.BlockSpec((R, dout), lambda i: (i, 0)),
        out_shape=jax.ShapeDtypeStruct((N, dout), jnp.float32),
    )(x, agg, deg, wself_t, wneigh_t, b)


def kernel(h, w, edge_index, W1, b1, W2, b2, W3, b3):
    src = edge_index[0].astype(jnp.int32)
    dst = edge_index[1].astype(jnp.int32)
    wf = w[:, 0]

    deg = _tc_deg(dst).reshape(NP, 1)
    agg1 = _sc_aggregate(src, dst, wf, h)
    x1 = _tc_dense(h, agg1, deg, W1[:, :D].T, W1[:, D:].T,
                   b1.reshape(1, -1), act=True)
    agg2 = _sc_aggregate(src, dst, wf, x1)
    x2 = _tc_dense(x1, agg2, deg, W2[:, :D].T, W2[:, D:].T,
                   b2.reshape(1, -1), act=True)
    agg3 = _sc_aggregate(src, dst, wf, x2)
    out = _tc_dense(x2, agg3, deg, W3[:, :D].T, W3[:, D:].T,
                    b3.reshape(1, -1), act=False)
    return out


# speedup vs baseline: 1.0760x; 1.0760x over previous
import functools

import numpy as np

import jax
import jax.numpy as jnp
from jax import lax
from jax.experimental import pallas as pl
from jax.experimental.pallas import tpu as pltpu
from jax.experimental.pallas import tpu_sc as plsc

N = 10000
E = 320000
D = 128

NC = 2
NS = 16
NP = 10240
HP = NP // NC
RPT = HP // NS
EPT = E // NS
C = 80
NCHUNK = EPT // C
ZR = 64
DR = HP // D

_PERM = np.zeros(D, np.int32)
for _b0 in range(0, D, 32):
    for _i in range(16):
        _PERM[_b0 + 2 * _i] = _b0 + _i
        _PERM[_b0 + 2 * _i + 1] = _b0 + 16 + _i

_GDN = lax.GatherDimensionNumbers(
    offset_dims=(), collapsed_slice_dims=(0,), start_index_map=(0,))


def _lane_bcast(v, j):
    idx = jnp.full((16, 1), j, dtype=jnp.int32)
    return lax.gather(v, idx, _GDN, (1,),
                      mode=lax.GatherScatterMode.PROMISE_IN_BOUNDS)


_lane_bcast_i32 = _lane_bcast


def _make_sc_aggregate(with_deg: bool):
    Cc = 80
    NCH = EPT // Cc
    NB = 2
    NMAIN = (NCH // NB) * NB

    mesh = plsc.VectorSubcoreMesh(core_axis_name="c", subcore_axis_name="s")
    out_type = jax.ShapeDtypeStruct((NP, D), jnp.float32)
    if with_deg:
        out_type = (out_type, jax.ShapeDtypeStruct((NC * DR, D), jnp.float32))
    scratch = [
        pltpu.VMEM((EPT,), jnp.int32),
        pltpu.VMEM((EPT,), jnp.float32),
        pltpu.VMEM((ZR, D), jnp.float32),
        pltpu.VMEM_SHARED((HP + 8, D), jnp.float32),
    ]
    for _ in range(NB):
        scratch += [
            pltpu.VMEM((Cc,), jnp.int32),
            pltpu.VMEM((Cc,), jnp.int32),
            pltpu.VMEM((Cc, D), jnp.float32),
            pltpu.SemaphoreType.DMA,
            pltpu.SemaphoreType.DMA,
            pltpu.SemaphoreType.DMA,
        ]
    if with_deg:
        scratch += [pltpu.VMEM_SHARED((DR + 8, D), jnp.float32)]
        for _ in range(NB):
            scratch += [
                pltpu.VMEM((Cc,), jnp.int32),
                pltpu.VMEM((Cc, D), jnp.float32),
            ]

    @functools.partial(pl.kernel, out_type=out_type, mesh=mesh,
                       scratch_types=scratch)
    def agg(src_hbm, dst_hbm, w_hbm, x_hbm, *rest):
        if with_deg:
            out_hbm, deg_hbm = rest[0], rest[1]
            rest = rest[2:]
        else:
            out_hbm = rest[0]
            rest = rest[1:]
        dbuf, wbig, zb = rest[0], rest[1], rest[2]
        acc = rest[3]
        slots = []
        k = 4
        for _ in range(NB):
            slots.append(rest[k:k + 6])
            k += 6
        if with_deg:
            dacc = rest[k]
            k += 1
            dslots = []
            for _ in range(NB):
                dslots.append(rest[k:k + 2])
                k += 2

        c = lax.axis_index("c")
        s = lax.axis_index("s")
        lo = c * HP
        ebase = s * EPT

        pltpu.sync_copy(dst_hbm.at[pl.ds(ebase, EPT)], dbuf)
        pltpu.sync_copy(w_hbm.at[pl.ds(ebase, EPT)], wbig)

        def fill(i, _):
            for q in range(D // 16):
                zb[i, pl.ds(q * 16, 16)] = jnp.zeros((16,), jnp.float32)
            return 0
        lax.fori_loop(0, ZR, fill, 0)

        base = pl.multiple_of(s * RPT, ZR)
        for k2 in range(RPT // ZR):
            r0 = pl.multiple_of(base + k2 * ZR, ZR)
            pltpu.sync_copy(zb, acc.at[pl.ds(r0, ZR)])
        if with_deg:
            @pl.when(s == 0)
            def _():
                pltpu.sync_copy(zb.at[pl.ds(0, DR + 8)], dacc)
        plsc.subcore_barrier()

        def eoff(t):
            return pl.multiple_of(ebase + t * Cc, 8)

        def issue_sidx(t, sl):
            pltpu.async_copy(src_hbm.at[pl.ds(eoff(t), Cc)], sl[0], sl[3])

        def wait_sidx(sl):
            pltpu.make_async_copy(src_hbm.at[pl.ds(ebase, Cc)], sl[0],
                                  sl[3]).wait()

        def issue_gather(sl):
            pltpu.async_copy(x_hbm.at[sl[0]], sl[2], sl[4])

        def wait_gather(sl):
            pltpu.make_async_copy(x_hbm.at[sl[0]], sl[2], sl[4]).wait()

        def issue_scatter(p, sl):
            pltpu.async_copy(sl[2], acc.at[sl[1]], sl[5], add=True)

        def wait_scatter(p, sl):
            pltpu.make_async_copy(sl[2], acc.at[sl[1]], sl[5]).wait()

        def scale(t, p, sl):
            toff = pl.multiple_of(t * Cc, 8)
            didx, rows = sl[1], sl[2]
            for g in range(Cc // 16):
                sl16 = pl.ds(g * 16, 16)
                goff = pl.ds(toff + g * 16, 16)
                rel = dbuf[goff] - lo
                inb = (rel >= 0) & (rel < HP)
                didx[sl16] = jnp.where(inb, rel, HP)
                wv = wbig[goff]
                if with_deg:
                    r2 = jnp.where(inb, rel, 0)
                    dslots[p][0][sl16] = jnp.where(inb, r2 >> 7, DR)
                    dlane = r2 & (D - 1)
                for j in range(16):
                    e = g * 16 + j
                    wb = _lane_bcast(wv, j)
                    for q in range(D // 16):
                        sq = pl.ds(q * 16, 16)
                        rows[e, sq] = rows[e, sq] * wb

        for tt in range(NB):
            issue_sidx(tt, slots[tt])
        wait_sidx(slots[0])
        issue_gather(slots[0])

        def body(i, _):
            for p in range(NB):
                t = NB * i + p
                pn = (p + 1) % NB
                wait_gather(slots[p])

                @pl.when(i + (1 if p >= 1 else 0) > 0)
                def _():
                    wait_scatter(pn, slots[pn])

                @pl.when(t + 1 < NCH)
                def _():
                    wait_sidx(slots[pn])
                    issue_gather(slots[pn])
                scale(t, p, slots[p])
                issue_scatter(p, slots[p])

                @pl.when(t + NB < NCH)
                def _():
                    issue_sidx(t + NB, slots[p])
            return 0
        lax.fori_loop(0, NMAIN // NB, body, 0)

        for t in range(NMAIN, NCH):
            p = t % NB
            pn = (p + 1) % NB
            wait_gather(slots[p])
            wait_scatter(pn, slots[pn])
            if t + 1 < NCH:
                wait_sidx(slots[pn])
                issue_gather(slots[pn])
            scale(t, p, slots[p])
            issue_scatter(p, slots[p])

        wait_scatter((NCH - 1) % NB, slots[(NCH - 1) % NB])
        plsc.subcore_barrier()

        for k3 in range(RPT // ZR):
            r0 = pl.multiple_of(base + k3 * ZR, ZR)
            go = lo + r0
            pltpu.sync_copy(acc.at[pl.ds(r0, ZR)], out_hbm.at[pl.ds(go, ZR)])
        if with_deg:
            @pl.when(s == 0)
            def _():
                pltpu.sync_copy(dacc.at[pl.ds(0, DR)],
                                deg_hbm.at[pl.ds(c * DR, DR)])

    return agg


_sc_aggregate_deg = _make_sc_aggregate(True)
_sc_aggregate = _make_sc_aggregate(False)

EB = 3200


def _tc_deg_body(dst_ref, o_ref):
    i = pl.program_id(0)

    @pl.when(i == 0)
    def _():
        o_ref[...] = jnp.zeros_like(o_ref)

    d = dst_ref[0, 0, :]
    dl = (d & (D - 1))[None, :]
    dr = (d >> 7)[None, :]
    iol = lax.broadcasted_iota(jnp.int32, (D, EB), 0)
    ior = lax.broadcasted_iota(jnp.int32, (NC * DR, EB), 0)
    lhs = jnp.maximum(1 - jnp.abs(ior - dr), 0).astype(jnp.bfloat16)
    rhs = jnp.maximum(1 - jnp.abs(iol - dl), 0).astype(jnp.bfloat16)
    o_ref[...] += lax.dot_general(
        lhs, rhs, (((1,), (1,)), ((), ())),
        preferred_element_type=jnp.float32)


def _tc_deg(dst):
    dst3 = dst.reshape(E // EB, 1, EB)
    return pl.pallas_call(
        _tc_deg_body,
        grid=(E // EB,),
        in_specs=[pl.BlockSpec((1, 1, EB), lambda i: (i, 0, 0))],
        out_specs=pl.BlockSpec((NC * DR, D), lambda i: (0, 0)),
        out_shape=jax.ShapeDtypeStruct((NC * DR, D), jnp.float32),
    )(dst3)


R = 1000


def _tc_dense_body(act, x_ref, agg_ref, deg_ref, ws_ref, wn_ref, b_ref, o_ref):
    h_n = agg_ref[...] / jnp.maximum(deg_ref[...], 1.0)
    y = (jnp.dot(x_ref[...], ws_ref[...], preferred_element_type=jnp.float32,
                 precision=lax.Precision.HIGHEST)
         + jnp.dot(h_n, wn_ref[...], preferred_element_type=jnp.float32,
                   precision=lax.Precision.HIGHEST)
         + b_ref[...])
    if act:
        y = jnp.maximum(y, 0.0)
        n2 = jnp.sum(y * y, axis=1, keepdims=True)
        y = y * lax.rsqrt(jnp.maximum(n2, 1e-24))
    o_ref[...] = y


def _tc_dense(x, agg, deg, wself_t, wneigh_t, b, act):
    dout = wself_t.shape[1]
    return pl.pallas_call(
        functools.partial(_tc_dense_body, act),
        grid=(N // R,),
        in_specs=[
            pl.BlockSpec((R, D), lambda i: (i, 0)),
            pl.BlockSpec((R, D), lambda i: (i, 0)),
            pl.BlockSpec((R, 1), lambda i: (i, 0)),
            pl.BlockSpec((D, dout), lambda i: (0, 0)),
            pl.BlockSpec((D, dout), lambda i: (0, 0)),
            pl.BlockSpec((1, dout), lambda i: (0, 0)),
        ],
        out_specs=pl.BlockSpec((R, dout), lambda i: (i, 0)),
        out_shape=jax.ShapeDtypeStruct((N, dout), jnp.float32),
    )(x, agg, deg, wself_t, wneigh_t, b)


_PERM_J = jnp.asarray(_PERM)


def _bf_table(x):
    return x[:, _PERM_J].astype(jnp.bfloat16).reshape(N, 2, 64)


def kernel(h, w, edge_index, W1, b1, W2, b2, W3, b3):
    src = edge_index[0].astype(jnp.int32)
    dst = edge_index[1].astype(jnp.int32)
    wf = w[:, 0]

    deg = _tc_deg(dst).reshape(NP, 1)
    agg1 = _sc_aggregate(src, dst, wf, h)
    x1 = _tc_dense(h, agg1, deg, W1[:, :D].T, W1[:, D:].T,
                   b1.reshape(1, -1), act=True)
    agg2 = _sc_aggregate(src, dst, wf, x1)
    x2 = _tc_dense(x1, agg2, deg, W2[:, :D].T, W2[:, D:].T,
                   b2.reshape(1, -1), act=True)
    agg3 = _sc_aggregate(src, dst, wf, x2)
    out = _tc_dense(x2, agg3, deg, W3[:, :D].T, W3[:, D:].T,
                    b3.reshape(1, -1), act=False)
    return out


# speedup vs baseline: 1.0764x; 1.0004x over previous
import functools

import jax
import jax.numpy as jnp
from jax import lax
from jax.experimental import pallas as pl
from jax.experimental.pallas import tpu as pltpu
from jax.experimental.pallas import tpu_sc as plsc

N = 10000
E = 320000
D = 128

NC = 2
NS = 16
NP = 10240
HP = NP // NC
RPT = HP // NS
EPT = E // NS
C = 80
NCHUNK = EPT // C
ZR = 64
DR = HP // D

_GDN = lax.GatherDimensionNumbers(
    offset_dims=(), collapsed_slice_dims=(0,), start_index_map=(0,))


def _lane_bcast(v, j):
    idx = jnp.full((16, 1), j, dtype=jnp.int32)
    return lax.gather(v, idx, _GDN, (1,),
                      mode=lax.GatherScatterMode.PROMISE_IN_BOUNDS)


_lane_bcast_i32 = _lane_bcast


def _make_sc_aggregate(with_deg: bool):
    Cc = 80
    NCH = EPT // Cc
    NB = 2
    NMAIN = (NCH // NB) * NB

    mesh = plsc.VectorSubcoreMesh(core_axis_name="c", subcore_axis_name="s",
                              num_cores=NC, num_subcores=NS)
    out_type = jax.ShapeDtypeStruct((NP, D), jnp.float32)
    if with_deg:
        out_type = (out_type, jax.ShapeDtypeStruct((NC * DR, D), jnp.float32))
    scratch = [
        pltpu.VMEM((EPT,), jnp.int32),
        pltpu.VMEM((EPT,), jnp.float32),
        pltpu.VMEM((ZR, D), jnp.float32),
        pltpu.VMEM_SHARED((HP + 8, D), jnp.float32),
    ]
    for _ in range(NB):
        scratch += [
            pltpu.VMEM((Cc,), jnp.int32),
            pltpu.VMEM((Cc,), jnp.int32),
            pltpu.VMEM((Cc, D), jnp.float32),
            pltpu.SemaphoreType.DMA,
            pltpu.SemaphoreType.DMA,
            pltpu.SemaphoreType.DMA,
        ]
    if with_deg:
        scratch += [pltpu.VMEM_SHARED((DR + 8, D), jnp.float32)]
        for _ in range(NB):
            scratch += [
                pltpu.VMEM((Cc,), jnp.int32),
                pltpu.VMEM((Cc, D), jnp.float32),
            ]

    @functools.partial(pl.kernel, out_type=out_type, mesh=mesh,
                       scratch_types=scratch)
    def agg(src_hbm, dst_hbm, w_hbm, x_hbm, *rest):
        if with_deg:
            out_hbm, deg_hbm = rest[0], rest[1]
            rest = rest[2:]
        else:
            out_hbm = rest[0]
            rest = rest[1:]
        dbuf, wbig, zb = rest[0], rest[1], rest[2]
        acc = rest[3]
        slots = []
        k = 4
        for _ in range(NB):
            slots.append(rest[k:k + 6])
            k += 6
        if with_deg:
            dacc = rest[k]
            k += 1
            dslots = []
            for _ in range(NB):
                dslots.append(rest[k:k + 2])
                k += 2

        c = lax.axis_index("c")
        s = lax.axis_index("s")
        lo = c * HP
        ebase = s * EPT

        pltpu.sync_copy(dst_hbm.at[pl.ds(ebase, EPT)], dbuf)
        pltpu.sync_copy(w_hbm.at[pl.ds(ebase, EPT)], wbig)

        def fill(i, _):
            for q in range(D // 16):
                zb[i, pl.ds(q * 16, 16)] = jnp.zeros((16,), jnp.float32)
            return 0
        lax.fori_loop(0, ZR, fill, 0)

        base = pl.multiple_of(s * RPT, ZR)
        for k2 in range(RPT // ZR):
            r0 = pl.multiple_of(base + k2 * ZR, ZR)
            pltpu.sync_copy(zb, acc.at[pl.ds(r0, ZR)])
        if with_deg:
            @pl.when(s == 0)
            def _():
                pltpu.sync_copy(zb.at[pl.ds(0, DR + 8)], dacc)
        plsc.subcore_barrier()

        def eoff(t):
            return pl.multiple_of(ebase + t * Cc, 8)

        def issue_sidx(t, sl):
            pltpu.async_copy(src_hbm.at[pl.ds(eoff(t), Cc)], sl[0], sl[3])

        def wait_sidx(sl):
            pltpu.make_async_copy(src_hbm.at[pl.ds(ebase, Cc)], sl[0],
                                  sl[3]).wait()

        def issue_gather(sl):
            pltpu.async_copy(x_hbm.at[sl[0]], sl[2], sl[4])

        def wait_gather(sl):
            pltpu.make_async_copy(x_hbm.at[sl[0]], sl[2], sl[4]).wait()

        def issue_scatter(p, sl):
            pltpu.async_copy(sl[2], acc.at[sl[1]], sl[5], add=True)

        def wait_scatter(p, sl):
            pltpu.make_async_copy(sl[2], acc.at[sl[1]], sl[5]).wait()

        def scale(t, p, sl):
            toff = pl.multiple_of(t * Cc, 8)
            didx, rows = sl[1], sl[2]
            for g in range(Cc // 16):
                sl16 = pl.ds(g * 16, 16)
                goff = pl.ds(toff + g * 16, 16)
                rel = dbuf[goff] - lo
                inb = (rel >= 0) & (rel < HP)
                didx[sl16] = jnp.where(inb, rel, HP)
                wv = wbig[goff]
                if with_deg:
                    r2 = jnp.where(inb, rel, 0)
                    dslots[p][0][sl16] = jnp.where(inb, r2 >> 7, DR)
                    dlane = r2 & (D - 1)
                for j in range(16):
                    e = g * 16 + j
                    wb = _lane_bcast(wv, j)
                    for q in range(D // 16):
                        sq = pl.ds(q * 16, 16)
                        rows[e, sq] = rows[e, sq] * wb

        for tt in range(NB):
            issue_sidx(tt, slots[tt])
        wait_sidx(slots[0])
        issue_gather(slots[0])

        def body(i, _):
            for p in range(NB):
                t = NB * i + p
                pn = (p + 1) % NB
                wait_gather(slots[p])

                @pl.when(i + (1 if p >= 1 else 0) > 0)
                def _():
                    wait_scatter(pn, slots[pn])

                @pl.when(t + 1 < NCH)
                def _():
                    wait_sidx(slots[pn])
                    issue_gather(slots[pn])
                scale(t, p, slots[p])
                issue_scatter(p, slots[p])

                @pl.when(t + NB < NCH)
                def _():
                    issue_sidx(t + NB, slots[p])
            return 0
        lax.fori_loop(0, NMAIN // NB, body, 0)

        for t in range(NMAIN, NCH):
            p = t % NB
            pn = (p + 1) % NB
            wait_gather(slots[p])
            wait_scatter(pn, slots[pn])
            if t + 1 < NCH:
                wait_sidx(slots[pn])
                issue_gather(slots[pn])
            scale(t, p, slots[p])
            issue_scatter(p, slots[p])

        wait_scatter((NCH - 1) % NB, slots[(NCH - 1) % NB])
        plsc.subcore_barrier()

        for k3 in range(RPT // ZR):
            r0 = pl.multiple_of(base + k3 * ZR, ZR)
            go = lo + r0
            pltpu.sync_copy(acc.at[pl.ds(r0, ZR)], out_hbm.at[pl.ds(go, ZR)])
        if with_deg:
            @pl.when(s == 0)
            def _():
                pltpu.sync_copy(dacc.at[pl.ds(0, DR)],
                                deg_hbm.at[pl.ds(c * DR, DR)])

    return agg


_sc_aggregate = _make_sc_aggregate(False)

EB = 3200


def _tc_deg_body(dst_ref, o_ref):
    i = pl.program_id(0)

    @pl.when(i == 0)
    def _():
        o_ref[...] = jnp.zeros_like(o_ref)

    d = dst_ref[0, 0, :]
    dl = (d & (D - 1))[None, :]
    dr = (d >> 7)[None, :]
    iol = lax.broadcasted_iota(jnp.int32, (D, EB), 0)
    ior = lax.broadcasted_iota(jnp.int32, (NC * DR, EB), 0)
    lhs = jnp.maximum(1 - jnp.abs(ior - dr), 0).astype(jnp.bfloat16)
    rhs = jnp.maximum(1 - jnp.abs(iol - dl), 0).astype(jnp.bfloat16)
    o_ref[...] += lax.dot_general(
        lhs, rhs, (((1,), (1,)), ((), ())),
        preferred_element_type=jnp.float32)


def _tc_deg(dst):
    dst3 = dst.reshape(E // EB, 1, EB)
    return pl.pallas_call(
        _tc_deg_body,
        grid=(E // EB,),
        in_specs=[pl.BlockSpec((1, 1, EB), lambda i: (i, 0, 0))],
        out_specs=pl.BlockSpec((NC * DR, D), lambda i: (0, 0)),
        out_shape=jax.ShapeDtypeStruct((NC * DR, D), jnp.float32),
    )(dst3)


R = 1000


def _tc_dense_body(act, x_ref, agg_ref, deg_ref, ws_ref, wn_ref, b_ref, o_ref):
    h_n = agg_ref[...] / jnp.maximum(deg_ref[...], 1.0)
    y = (jnp.dot(x_ref[...], ws_ref[...], preferred_element_type=jnp.float32,
                 precision=lax.Precision.HIGHEST)
         + jnp.dot(h_n, wn_ref[...], preferred_element_type=jnp.float32,
                   precision=lax.Precision.HIGHEST)
         + b_ref[...])
    if act:
        y = jnp.maximum(y, 0.0)
        n2 = jnp.sum(y * y, axis=1, keepdims=True)
        y = y * lax.rsqrt(jnp.maximum(n2, 1e-24))
    o_ref[...] = y


def _tc_dense(x, agg, deg, wself_t, wneigh_t, b, act):
    dout = wself_t.shape[1]
    return pl.pallas_call(
        functools.partial(_tc_dense_body, act),
        grid=(N // R,),
        in_specs=[
            pl.BlockSpec((R, D), lambda i: (i, 0)),
            pl.BlockSpec((R, D), lambda i: (i, 0)),
            pl.BlockSpec((R, 1), lambda i: (i, 0)),
            pl.BlockSpec((D, dout), lambda i: (0, 0)),
            pl.BlockSpec((D, dout), lambda i: (0, 0)),
            pl.BlockSpec((1, dout), lambda i: (0, 0)),
        ],
        out_specs=pl.BlockSpec((R, dout), lambda i: (i, 0)),
        out_shape=jax.ShapeDtypeStruct((N, dout), jnp.float32),
    )(x, agg, deg, wself_t, wneigh_t, b)


def kernel(h, w, edge_index, W1, b1, W2, b2, W3, b3):
    src = edge_index[0].astype(jnp.int32)
    dst = edge_index[1].astype(jnp.int32)
    wf = w[:, 0]

    deg = _tc_deg(dst).reshape(NP, 1)
    agg1 = _sc_aggregate(src, dst, wf, h)
    x1 = _tc_dense(h, agg1, deg, W1[:, :D].T, W1[:, D:].T,
                   b1.reshape(1, -1), act=True)
    agg2 = _sc_aggregate(src, dst, wf, x1)
    x2 = _tc_dense(x1, agg2, deg, W2[:, :D].T, W2[:, D:].T,
                   b2.reshape(1, -1), act=True)
    agg3 = _sc_aggregate(src, dst, wf, x2)
    out = _tc_dense(x2, agg3, deg, W3[:, :D].T, W3[:, D:].T,
                    b3.reshape(1, -1), act=False)
    return out
